# Initial kernel scaffold; baseline (speedup 1.0000x reference)
import jax
import jax.numpy as jnp
from jax.experimental import pallas as pl


def kernel(X, edge_index, batch, params):
    raise NotImplementedError("write your pallas kernel here")


# speedup vs baseline: 2.3739x; 2.3739x over previous
import functools

import jax
import jax.numpy as jnp
from jax import lax
from jax.experimental import pallas as pl
from jax.experimental.pallas import tpu as pltpu
from jax.experimental.pallas import tpu_sc as plsc

N = 10000
E = 320000
H = 128
G = 128

NC = 2
NS = 16
NT = NC * NS
C = 48
CH_PER_TILE = 216
GRP = 8
NGRP = CH_PER_TILE // GRP
EPAD = NT * CH_PER_TILE * C
NPAD = 10240
ROWS_PER_TILE = NPAD // NS
ZROWS = 32


def _dot(a, b):
    return jnp.dot(a, b, preferred_element_type=jnp.float32)


def _edge_body(k_hbm, q_hbm, v_hbm, src_hbm, dst_hbm, out_hbm,
               srcg, dstg, kd0, kd1, eq0, eq1, vv0, vv1, msgv,
               aggs, semi, semg0, semg1):
    cid = lax.axis_index("c")
    sid = lax.axis_index("s")
    wid = cid * NS + sid

    def _zrow(i, _):
        for hh in range(H // 16):
            kd0[i, pl.ds(hh * 16, 16)] = jnp.zeros((16,), jnp.float32)
        return 0
    lax.fori_loop(0, ZROWS, _zrow, 0)
    for r in range(ROWS_PER_TILE // ZROWS):
        pltpu.sync_copy(kd0.at[pl.ds(0, ZROWS)],
                        aggs.at[pl.ds(sid * ROWS_PER_TILE + r * ZROWS, ZROWS)])
    plsc.subcore_barrier()

    tbase = wid * CH_PER_TILE
    pltpu.async_copy(src_hbm.at[pl.ds(tbase, GRP)], srcg.at[0], semi)
    pltpu.async_copy(dst_hbm.at[pl.ds(tbase, GRP)], dstg.at[0], semi)

    kbuf = (kd0, kd1)
    qbuf = (eq0, eq1)
    vbuf = (vv0, vv1)
    gsem = (semg0, semg1)

    def _issue(gm, jj, b):
        hk = pltpu.async_copy(k_hbm.at[dstg.at[gm, jj]], kbuf[b], gsem[b])
        hq = pltpu.async_copy(q_hbm.at[srcg.at[gm, jj]], qbuf[b], gsem[b])
        hv = pltpu.async_copy(v_hbm.at[srcg.at[gm, jj]], vbuf[b], gsem[b])
        return (hk, hq, hv)

    def _group(g, _):
        gm = lax.rem(g, 2)
        gbase = tbase + g * GRP
        pltpu.make_async_copy(src_hbm.at[pl.ds(gbase, GRP)],
                              srcg.at[gm], semi).wait()
        pltpu.make_async_copy(dst_hbm.at[pl.ds(gbase, GRP)],
                              dstg.at[gm], semi).wait()

        @pl.when(g + 1 < NGRP)
        def _():
            nb = gbase + GRP
            pltpu.async_copy(src_hbm.at[pl.ds(nb, GRP)], srcg.at[1 - gm],
                             semi)
            pltpu.async_copy(dst_hbm.at[pl.ds(nb, GRP)], dstg.at[1 - gm],
                             semi)

        pend = _issue(gm, 0, 0)
        for jj in range(GRP):
            b = jj % 2
            for hc in pend:
                hc.wait()
            if jj < GRP - 1:
                pend = _issue(gm, jj + 1, 1 - b)
            kb, qb, vb = kbuf[b], qbuf[b], vbuf[b]

            def _edge(e, _, kb=kb, qb=qb, vb=vb):
                nsl = H // 16
                eks = [kb[e, pl.ds(i * 16, 16)] for i in range(nsl)]
                eqs = [qb[e, pl.ds(i * 16, 16)] for i in range(nsl)]
                vss = [vb[e, pl.ds(i * 16, 16)] for i in range(nsl)]
                ds = [1.0 + eks[i] * eqs[i] for i in range(nsl)]
                rs = [1.0 / ds[i] for i in range(nsl)]
                ms = [rs[i] * (2.0 - ds[i] * rs[i]) * vss[i]
                      for i in range(nsl)]
                for i in range(nsl):
                    msgv[e, pl.ds(i * 16, 16)] = ms[i]
                return 0
            lax.fori_loop(0, C, _edge, 0)

            pltpu.sync_copy(msgv, aggs.at[dstg.at[gm, jj]], add=True)
        return 0
    lax.fori_loop(0, NGRP, _group, 0)

    plsc.subcore_barrier()
    pltpu.sync_copy(aggs.at[pl.ds(sid * ROWS_PER_TILE, ROWS_PER_TILE)],
                    out_hbm.at[cid, pl.ds(sid * ROWS_PER_TILE, ROWS_PER_TILE)])


_edge_call = functools.partial(
    pl.kernel,
    mesh=plsc.VectorSubcoreMesh(core_axis_name="c", subcore_axis_name="s"),
    out_type=jax.ShapeDtypeStruct((NC, NPAD, H), jnp.float32),
    scratch_types=[
        pltpu.VMEM((2, GRP, C), jnp.int32),
        pltpu.VMEM((2, GRP, C), jnp.int32),
        pltpu.VMEM((C, H), jnp.float32),
        pltpu.VMEM((C, H), jnp.float32),
        pltpu.VMEM((C, H), jnp.float32),
        pltpu.VMEM((C, H), jnp.float32),
        pltpu.VMEM((C, H), jnp.float32),
        pltpu.VMEM((C, H), jnp.float32),
        pltpu.VMEM((C, H), jnp.float32),
        pltpu.VMEM_SHARED((NPAD, H), jnp.float32),
        pltpu.SemaphoreType.DMA,
        pltpu.SemaphoreType.DMA,
        pltpu.SemaphoreType.DMA,
    ],
)(_edge_body)


R = 1000
NB = N // R


def _negexp(x):
    return jnp.exp(-jnp.clip(x, -44.0, 44.0))


def _mm4_body(x_ref, w_ref, b_ref, k_ref, q_ref, v_ref, s_ref):
    acc = _dot(x_ref[...], w_ref[...]) + b_ref[...]
    k_ref[...] = _negexp(acc[:, :H])
    q_ref[...] = _negexp(acc[:, H:2 * H])
    v_ref[...] = acc[:, 2 * H:3 * H]
    s_ref[...] = acc[:, 3 * H:]


_MM4_OUT_SPECS = [
    pl.BlockSpec((R, H), lambda i: (i, 0)),
    pl.BlockSpec((R, H), lambda i: (i, 0)),
    pl.BlockSpec((R, H), lambda i: (i, 0)),
    pl.BlockSpec((R, H), lambda i: (i, 0)),
]
_MM4_OUT_SHAPE = [
    jax.ShapeDtypeStruct((NPAD, H), jnp.float32),
    jax.ShapeDtypeStruct((NPAD, H), jnp.float32),
    jax.ShapeDtypeStruct((NPAD, H), jnp.float32),
    jax.ShapeDtypeStruct((N, H), jnp.float32),
]


def _mm4(x, w4, b4):
    return pl.pallas_call(
        _mm4_body,
        grid=(NB,),
        in_specs=[
            pl.BlockSpec((R, H), lambda i: (i, 0)),
            pl.BlockSpec((H, 4 * H), lambda i: (0, 0)),
            pl.BlockSpec((1, 4 * H), lambda i: (0, 0)),
        ],
        out_specs=_MM4_OUT_SPECS,
        out_shape=_MM4_OUT_SHAPE,
    )(x, w4, b4)


def _relu_stats_body(a_ref, s_ref, y_ref, st_ref):
    i = pl.program_id(0)
    y = jnp.maximum(a_ref[0] + a_ref[1] + s_ref[...], 0.0)
    y_ref[...] = y
    part = jnp.concatenate(
        [jnp.sum(y, axis=0, keepdims=True),
         jnp.sum(y * y, axis=0, keepdims=True)], axis=0)

    @pl.when(i == 0)
    def _():
        st_ref[...] = part

    @pl.when(i > 0)
    def _():
        st_ref[...] = st_ref[...] + part


def _relu_stats(agg2, s):
    return pl.pallas_call(
        _relu_stats_body,
        grid=(NB,),
        in_specs=[
            pl.BlockSpec((NC, R, H), lambda i: (0, i, 0)),
            pl.BlockSpec((R, H), lambda i: (i, 0)),
        ],
        out_specs=[
            pl.BlockSpec((R, H), lambda i: (i, 0)),
            pl.BlockSpec((2, H), lambda i: (0, 0)),
        ],
        out_shape=[
            jax.ShapeDtypeStruct((N, H), jnp.float32),
            jax.ShapeDtypeStruct((2, H), jnp.float32),
        ],
    )(agg2, s)


def _bn(y, st_ref, g_ref, b_ref):
    mu = st_ref[0:1] * (1.0 / N)
    var = st_ref[1:2] * (1.0 / N) - mu * mu
    inv = jax.lax.rsqrt(var + 1e-5)
    return (y - mu) * inv * g_ref[...] + b_ref[...]


def _bn_mm4_body(y_ref, st_ref, g_ref, b_ref, w_ref, bb_ref,
                 k_ref, q_ref, v_ref, s_ref):
    h = _bn(y_ref[...], st_ref, g_ref, b_ref)
    acc = _dot(h, w_ref[...]) + bb_ref[...]
    k_ref[...] = _negexp(acc[:, :H])
    q_ref[...] = _negexp(acc[:, H:2 * H])
    v_ref[...] = acc[:, 2 * H:3 * H]
    s_ref[...] = acc[:, 3 * H:]


def _bn_mm4(y, st, g, b, w4, b4):
    return pl.pallas_call(
        _bn_mm4_body,
        grid=(NB,),
        in_specs=[
            pl.BlockSpec((R, H), lambda i: (i, 0)),
            pl.BlockSpec((2, H), lambda i: (0, 0)),
            pl.BlockSpec((1, H), lambda i: (0, 0)),
            pl.BlockSpec((1, H), lambda i: (0, 0)),
            pl.BlockSpec((H, 4 * H), lambda i: (0, 0)),
            pl.BlockSpec((1, 4 * H), lambda i: (0, 0)),
        ],
        out_specs=_MM4_OUT_SPECS,
        out_shape=_MM4_OUT_SHAPE,
    )(y, st, g, b, w4, b4)


def _bn_gsum_body(y_ref, st_ref, g_ref, b_ref, bat_ref, sum_ref, cnt_ref):
    i = pl.program_id(0)
    h = _bn(y_ref[...], st_ref, g_ref, b_ref)
    gids = jax.lax.broadcasted_iota(jnp.int32, (G, R), 0)
    onehot = jnp.where(gids == bat_ref[0], 1.0, 0.0)
    psum = _dot(onehot, h)
    pcnt = _dot(onehot, jnp.ones((R, 1), jnp.float32))

    @pl.when(i == 0)
    def _():
        sum_ref[...] = psum
        cnt_ref[...] = pcnt

    @pl.when(i > 0)
    def _():
        sum_ref[...] = sum_ref[...] + psum
        cnt_ref[...] = cnt_ref[...] + pcnt


def _bn_gsum(y, st, g, b, batch_row):
    return pl.pallas_call(
        _bn_gsum_body,
        grid=(NB,),
        in_specs=[
            pl.BlockSpec((R, H), lambda i: (i, 0)),
            pl.BlockSpec((2, H), lambda i: (0, 0)),
            pl.BlockSpec((1, H), lambda i: (0, 0)),
            pl.BlockSpec((1, H), lambda i: (0, 0)),
            pl.BlockSpec((1, 1, R), lambda i: (i, 0, 0)),
        ],
        out_specs=[
            pl.BlockSpec((G, H), lambda i: (0, 0)),
            pl.BlockSpec((G, 1), lambda i: (0, 0)),
        ],
        out_shape=[
            jax.ShapeDtypeStruct((G, H), jnp.float32),
            jax.ShapeDtypeStruct((G, 1), jnp.float32),
        ],
    )(y, st, g, b, batch_row)


def _head_body(sum_ref, cnt_ref, wft_ref, bft_ref, lng_ref, lnb_ref,
               wa1_ref, ba1_ref, wa2_ref, ba2_ref, wr_ref, br_ref, out_ref):
    xm = sum_ref[...] / jnp.maximum(cnt_ref[...], 1.0)
    t = _dot(xm, wft_ref[...]) + bft_ref[...]
    mu = jnp.mean(t, axis=1, keepdims=True)
    var = jnp.mean(t * t, axis=1, keepdims=True) - mu * mu
    t = (t - mu) * jax.lax.rsqrt(var + 1e-5) * lng_ref[...] + lnb_ref[...]
    t = jnp.maximum(t, 0.0)
    a = jnp.maximum(_dot(t, wa1_ref[...]) + ba1_ref[...], 0.0)
    att = jax.nn.sigmoid(_dot(a, wa2_ref[...]) + ba2_ref[...])
    out_ref[...] = _dot(t * att, wr_ref[...]) + br_ref[...]


def _head(sums, cnts, hp):
    return pl.pallas_call(
        _head_body,
        out_shape=jax.ShapeDtypeStruct((G, 1), jnp.float32),
    )(sums, cnts,
      hp["W_ft"], hp["b_ft"].reshape(1, H),
      hp["ln_g"].reshape(1, H), hp["ln_b"].reshape(1, H),
      hp["Wa1"], hp["ba1"].reshape(1, H // 4),
      hp["Wa2"], hp["ba2"].reshape(1, 1),
      hp["Wr"], hp["br"].reshape(1, 1))


def kernel(X, edge_index, batch, params):
    pad = jnp.full((EPAD - E,), N, dtype=jnp.int32)
    src2 = jnp.concatenate([edge_index[0], pad]).reshape(NT * CH_PER_TILE, C)
    dst2 = jnp.concatenate([edge_index[1], pad]).reshape(NT * CH_PER_TILE, C)
    batch_row = batch.reshape(NB, 1, R)

    w4, b4 = [], []
    for cv in params["convs"]:
        w4.append(jnp.concatenate(
            [cv["Wk"], cv["Wq"], cv["Wv"], cv["Ws"]], axis=1))
        b4.append(jnp.concatenate(
            [cv["bk"], cv["bq"], cv["bv"], cv["bs"]]).reshape(1, 4 * H))
    gam = [b["gamma"].reshape(1, H) for b in params["bns"]]
    bet = [b["beta"].reshape(1, H) for b in params["bns"]]

    ek, eq, vv, s = _mm4(X, w4[0], b4[0])
    for i in range(5):
        agg2 = _edge_call(ek, eq, vv, src2, dst2)
        y, st = _relu_stats(agg2, s)
        if i < 4:
            ek, eq, vv, s = _bn_mm4(y, st, gam[i], bet[i], w4[i + 1], b4[i + 1])
        else:
            sums, cnts = _bn_gsum(y, st, gam[i], bet[i], batch_row)
    return _head(sums, cnts, params["head"])


# speedup vs baseline: 7.7666x; 3.2717x over previous
import functools

import jax
import jax.numpy as jnp
from jax import lax
from jax.experimental import pallas as pl
from jax.experimental.pallas import tpu as pltpu
from jax.experimental.pallas import tpu_sc as plsc

N = 10000
E = 320000
H = 128
G = 128

NC = 2
NS = 16
NT = NC * NS
C = 48
CH_PER_TILE = 216
GRP = 8
NGRP = CH_PER_TILE // GRP
EPAD = NT * CH_PER_TILE * C
NPAD = 10240
ROWS_PER_TILE = NPAD // NS
ZROWS = 32


def _dot(a, b):
    return jnp.dot(a, b, preferred_element_type=jnp.float32)


def _edge_body(k_hbm, q_hbm, v_hbm, src_hbm, dst_hbm, out_hbm,
               srcg, dstg, kd0, kd1, eq0, eq1, vv0, vv1, msgv,
               aggs, semi, semg0, semg1):
    cid = lax.axis_index("c")
    sid = lax.axis_index("s")
    wid = cid * NS + sid

    def _zrow(i, _):
        for hh in range(H // 16):
            kd0[i, pl.ds(hh * 16, 16)] = jnp.zeros((16,), jnp.float32)
        return 0
    lax.fori_loop(0, ZROWS, _zrow, 0)
    for r in range(ROWS_PER_TILE // ZROWS):
        pltpu.sync_copy(kd0.at[pl.ds(0, ZROWS)],
                        aggs.at[pl.ds(sid * ROWS_PER_TILE + r * ZROWS, ZROWS)])
    plsc.subcore_barrier()

    tbase = wid * CH_PER_TILE
    pltpu.async_copy(src_hbm.at[pl.ds(tbase, GRP)], srcg.at[0], semi)
    pltpu.async_copy(dst_hbm.at[pl.ds(tbase, GRP)], dstg.at[0], semi)

    kbuf = (kd0, kd1)
    qbuf = (eq0, eq1)
    vbuf = (vv0, vv1)
    gsem = (semg0, semg1)

    def _issue(gm, jj, b):
        hk = pltpu.async_copy(k_hbm.at[dstg.at[gm, jj]], kbuf[b], gsem[b])
        hq = pltpu.async_copy(q_hbm.at[srcg.at[gm, jj]], qbuf[b], gsem[b])
        hv = pltpu.async_copy(v_hbm.at[srcg.at[gm, jj]], vbuf[b], gsem[b])
        return (hk, hq, hv)

    def _group(g, _):
        gm = lax.rem(g, 2)
        gbase = tbase + g * GRP
        pltpu.make_async_copy(src_hbm.at[pl.ds(gbase, GRP)],
                              srcg.at[gm], semi).wait()
        pltpu.make_async_copy(dst_hbm.at[pl.ds(gbase, GRP)],
                              dstg.at[gm], semi).wait()

        @pl.when(g + 1 < NGRP)
        def _():
            nb = gbase + GRP
            pltpu.async_copy(src_hbm.at[pl.ds(nb, GRP)], srcg.at[1 - gm],
                             semi)
            pltpu.async_copy(dst_hbm.at[pl.ds(nb, GRP)], dstg.at[1 - gm],
                             semi)

        pend = _issue(gm, 0, 0)
        for jj in range(GRP):
            b = jj % 2
            for hc in pend:
                hc.wait()
            if jj < GRP - 1:
                pend = _issue(gm, jj + 1, 1 - b)
            kb, qb, vb = kbuf[b], qbuf[b], vbuf[b]

            def _edge(e, _, kb=kb, qb=qb, vb=vb):
                nsl = H // 16
                eks = [kb[e, pl.ds(i * 16, 16)] for i in range(nsl)]
                eqs = [qb[e, pl.ds(i * 16, 16)] for i in range(nsl)]
                vss = [vb[e, pl.ds(i * 16, 16)] for i in range(nsl)]
                ds = [1.0 + eks[i] * eqs[i] for i in range(nsl)]
                rs = [1.0 / ds[i] for i in range(nsl)]
                ms = [rs[i] * (2.0 - ds[i] * rs[i]) * vss[i]
                      for i in range(nsl)]
                for i in range(nsl):
                    msgv[e, pl.ds(i * 16, 16)] = ms[i]
                return 0
            lax.fori_loop(0, C, _edge, 0)

            pltpu.sync_copy(msgv, aggs.at[dstg.at[gm, jj]], add=True)
        return 0
    lax.fori_loop(0, NGRP, _group, 0)

    plsc.subcore_barrier()
    pltpu.sync_copy(aggs.at[pl.ds(sid * ROWS_PER_TILE, ROWS_PER_TILE)],
                    out_hbm.at[cid, pl.ds(sid * ROWS_PER_TILE, ROWS_PER_TILE)])


_edge_call = functools.partial(
    pl.kernel,
    mesh=plsc.VectorSubcoreMesh(core_axis_name="c", subcore_axis_name="s"),
    out_type=jax.ShapeDtypeStruct((NC, NPAD, H), jnp.float32),
    scratch_types=[
        pltpu.VMEM((2, GRP, C), jnp.int32),
        pltpu.VMEM((2, GRP, C), jnp.int32),
        pltpu.VMEM((C, H), jnp.float32),
        pltpu.VMEM((C, H), jnp.float32),
        pltpu.VMEM((C, H), jnp.float32),
        pltpu.VMEM((C, H), jnp.float32),
        pltpu.VMEM((C, H), jnp.float32),
        pltpu.VMEM((C, H), jnp.float32),
        pltpu.VMEM((C, H), jnp.float32),
        pltpu.VMEM_SHARED((NPAD, H), jnp.float32),
        pltpu.SemaphoreType.DMA,
        pltpu.SemaphoreType.DMA,
        pltpu.SemaphoreType.DMA,
    ],
)(_edge_body)


R = 1000
NB = N // R


def _negexp(x):
    return jnp.exp(-jnp.clip(x, -44.0, 44.0))


def _emit_kqvs(i, acc, k_ref, q_ref, v_ref, s_ref):
    @pl.when(i < NB)
    def _():
        k_ref[...] = _negexp(acc[:, :H])
        q_ref[...] = _negexp(acc[:, H:2 * H])
        v_ref[...] = acc[:, 2 * H:3 * H]
        s_ref[...] = acc[:, 3 * H:]

    @pl.when(i == NB)
    def _():
        k_ref[...] = jnp.ones(k_ref.shape, k_ref.dtype)
        q_ref[...] = jnp.ones(q_ref.shape, q_ref.dtype)
        v_ref[...] = jnp.zeros(v_ref.shape, v_ref.dtype)
        s_ref[...] = jnp.zeros(s_ref.shape, s_ref.dtype)


def _mm4_body(x_ref, w_ref, b_ref, k_ref, q_ref, v_ref, s_ref):
    i = pl.program_id(0)
    acc = _dot(x_ref[...], w_ref[...]) + b_ref[...]
    _emit_kqvs(i, acc, k_ref, q_ref, v_ref, s_ref)


def _clampi(i):
    return jnp.minimum(i, NB - 1)


_MM4_OUT_SPECS = [
    pl.BlockSpec((R, H), lambda i: (i, 0)),
    pl.BlockSpec((R, H), lambda i: (i, 0)),
    pl.BlockSpec((R, H), lambda i: (i, 0)),
    pl.BlockSpec((R, H), lambda i: (i, 0)),
]
_MM4_OUT_SHAPE = [
    jax.ShapeDtypeStruct((NPAD, H), jnp.float32),
    jax.ShapeDtypeStruct((NPAD, H), jnp.float32),
    jax.ShapeDtypeStruct((NPAD, H), jnp.float32),
    jax.ShapeDtypeStruct((NPAD, H), jnp.float32),
]


def _mm4(x, w4, b4):
    return pl.pallas_call(
        _mm4_body,
        grid=(NB + 1,),
        in_specs=[
            pl.BlockSpec((R, H), lambda i: (_clampi(i), 0)),
            pl.BlockSpec((H, 4 * H), lambda i: (0, 0)),
            pl.BlockSpec((1, 4 * H), lambda i: (0, 0)),
        ],
        out_specs=_MM4_OUT_SPECS,
        out_shape=_MM4_OUT_SHAPE,
    )(x, w4, b4)


def _relu_stats_body(a_ref, s_ref, y_ref, st_ref):
    i = pl.program_id(0)
    y = jnp.maximum(a_ref[0] + a_ref[1] + s_ref[...], 0.0)
    y_ref[...] = y
    part = jnp.concatenate(
        [jnp.sum(y, axis=0, keepdims=True),
         jnp.sum(y * y, axis=0, keepdims=True)], axis=0)

    @pl.when(i == 0)
    def _():
        st_ref[...] = part

    @pl.when(i > 0)
    def _():
        st_ref[...] = st_ref[...] + part


def _relu_stats(agg2, s):
    return pl.pallas_call(
        _relu_stats_body,
        grid=(NB,),
        in_specs=[
            pl.BlockSpec((NC, R, H), lambda i: (0, i, 0)),
            pl.BlockSpec((R, H), lambda i: (i, 0)),
        ],
        out_specs=[
            pl.BlockSpec((R, H), lambda i: (i, 0)),
            pl.BlockSpec((2, H), lambda i: (0, 0)),
        ],
        out_shape=[
            jax.ShapeDtypeStruct((N, H), jnp.float32),
            jax.ShapeDtypeStruct((2, H), jnp.float32),
        ],
    )(agg2, s)


def _bn(y, st_ref, g_ref, b_ref):
    mu = st_ref[0:1] * (1.0 / N)
    var = st_ref[1:2] * (1.0 / N) - mu * mu
    inv = jax.lax.rsqrt(var + 1e-5)
    return (y - mu) * inv * g_ref[...] + b_ref[...]


def _bn_mm4_body(y_ref, st_ref, g_ref, b_ref, w_ref, bb_ref,
                 k_ref, q_ref, v_ref, s_ref):
    i = pl.program_id(0)
    h = _bn(y_ref[...], st_ref, g_ref, b_ref)
    acc = _dot(h, w_ref[...]) + bb_ref[...]
    _emit_kqvs(i, acc, k_ref, q_ref, v_ref, s_ref)


def _bn_mm4(y, st, g, b, w4, b4):
    return pl.pallas_call(
        _bn_mm4_body,
        grid=(NB + 1,),
        in_specs=[
            pl.BlockSpec((R, H), lambda i: (_clampi(i), 0)),
            pl.BlockSpec((2, H), lambda i: (0, 0)),
            pl.BlockSpec((1, H), lambda i: (0, 0)),
            pl.BlockSpec((1, H), lambda i: (0, 0)),
            pl.BlockSpec((H, 4 * H), lambda i: (0, 0)),
            pl.BlockSpec((1, 4 * H), lambda i: (0, 0)),
        ],
        out_specs=_MM4_OUT_SPECS,
        out_shape=_MM4_OUT_SHAPE,
    )(y, st, g, b, w4, b4)


def _bn_gsum_body(y_ref, st_ref, g_ref, b_ref, bat_ref, sum_ref, cnt_ref):
    i = pl.program_id(0)
    h = _bn(y_ref[...], st_ref, g_ref, b_ref)
    gids = jax.lax.broadcasted_iota(jnp.int32, (G, R), 0)
    onehot = jnp.where(gids == bat_ref[0], 1.0, 0.0)
    psum = _dot(onehot, h)
    pcnt = _dot(onehot, jnp.ones((R, 1), jnp.float32))

    @pl.when(i == 0)
    def _():
        sum_ref[...] = psum
        cnt_ref[...] = pcnt

    @pl.when(i > 0)
    def _():
        sum_ref[...] = sum_ref[...] + psum
        cnt_ref[...] = cnt_ref[...] + pcnt


def _bn_gsum(y, st, g, b, batch_row):
    return pl.pallas_call(
        _bn_gsum_body,
        grid=(NB,),
        in_specs=[
            pl.BlockSpec((R, H), lambda i: (i, 0)),
            pl.BlockSpec((2, H), lambda i: (0, 0)),
            pl.BlockSpec((1, H), lambda i: (0, 0)),
            pl.BlockSpec((1, H), lambda i: (0, 0)),
            pl.BlockSpec((1, 1, R), lambda i: (i, 0, 0)),
        ],
        out_specs=[
            pl.BlockSpec((G, H), lambda i: (0, 0)),
            pl.BlockSpec((G, 1), lambda i: (0, 0)),
        ],
        out_shape=[
            jax.ShapeDtypeStruct((G, H), jnp.float32),
            jax.ShapeDtypeStruct((G, 1), jnp.float32),
        ],
    )(y, st, g, b, batch_row)


def _head_body(sum_ref, cnt_ref, wft_ref, bft_ref, lng_ref, lnb_ref,
               wa1_ref, ba1_ref, wa2_ref, ba2_ref, wr_ref, br_ref, out_ref):
    xm = sum_ref[...] / jnp.maximum(cnt_ref[...], 1.0)
    t = _dot(xm, wft_ref[...]) + bft_ref[...]
    mu = jnp.mean(t, axis=1, keepdims=True)
    var = jnp.mean(t * t, axis=1, keepdims=True) - mu * mu
    t = (t - mu) * jax.lax.rsqrt(var + 1e-5) * lng_ref[...] + lnb_ref[...]
    t = jnp.maximum(t, 0.0)
    a = jnp.maximum(_dot(t, wa1_ref[...]) + ba1_ref[...], 0.0)
    att = jax.nn.sigmoid(_dot(a, wa2_ref[...]) + ba2_ref[...])
    out_ref[...] = _dot(t * att, wr_ref[...]) + br_ref[...]


def _head(sums, cnts, hp):
    return pl.pallas_call(
        _head_body,
        out_shape=jax.ShapeDtypeStruct((G, 1), jnp.float32),
    )(sums, cnts,
      hp["W_ft"], hp["b_ft"].reshape(1, H),
      hp["ln_g"].reshape(1, H), hp["ln_b"].reshape(1, H),
      hp["Wa1"], hp["ba1"].reshape(1, H // 4),
      hp["Wa2"], hp["ba2"].reshape(1, 1),
      hp["Wr"], hp["br"].reshape(1, 1))


def kernel(X, edge_index, batch, params):
    pad = N + (jnp.arange(EPAD - E, dtype=jnp.int32) % (NPAD - N))
    src2 = jnp.concatenate([edge_index[0], pad]).reshape(NT * CH_PER_TILE, C)
    dst2 = jnp.concatenate([edge_index[1], pad]).reshape(NT * CH_PER_TILE, C)
    batch_row = batch.reshape(NB, 1, R)

    w4, b4 = [], []
    for cv in params["convs"]:
        w4.append(jnp.concatenate(
            [cv["Wk"], cv["Wq"], cv["Wv"], cv["Ws"]], axis=1))
        b4.append(jnp.concatenate(
            [cv["bk"], cv["bq"], cv["bv"], cv["bs"]]).reshape(1, 4 * H))
    gam = [b["gamma"].reshape(1, H) for b in params["bns"]]
    bet = [b["beta"].reshape(1, H) for b in params["bns"]]

    ek, eq, vv, s = _mm4(X, w4[0], b4[0])
    for i in range(5):
        agg2 = _edge_call(ek, eq, vv, src2, dst2)
        y, st = _relu_stats(agg2, s)
        if i < 4:
            ek, eq, vv, s = _bn_mm4(y, st, gam[i], bet[i], w4[i + 1], b4[i + 1])
        else:
            sums, cnts = _bn_gsum(y, st, gam[i], bet[i], batch_row)
    return _head(sums, cnts, params["head"])


# speedup vs baseline: 10.3680x; 1.3349x over previous
import functools

import jax
import jax.numpy as jnp
from jax import lax
from jax.experimental import pallas as pl
from jax.experimental.pallas import tpu as pltpu
from jax.experimental.pallas import tpu_sc as plsc

N = 10000
E = 320000
H = 128
G = 128

NC = 2
NS = 16
NT = NC * NS
C = 64
CH_PER_TILE = 160
GRP = 8
NGRP = CH_PER_TILE // GRP
EPAD = NT * CH_PER_TILE * C
NPAD = 10240
ROWS_PER_TILE = NPAD // NS
ZROWS = 32


def _dot(a, b):
    return jnp.dot(a, b, preferred_element_type=jnp.float32)


def _edge_body(k_hbm, qv_hbm, src_hbm, dst_hbm, out_hbm,
               srcg, dstg, kd0, kd1, qv0, qv1, msgv,
               aggs, semi, semg0, semg1):
    cid = lax.axis_index("c")
    sid = lax.axis_index("s")
    wid = cid * NS + sid

    def _zrow(i, _):
        for hh in range(H // 16):
            kd0[i, pl.ds(hh * 16, 16)] = jnp.zeros((16,), jnp.float32)
        return 0
    lax.fori_loop(0, ZROWS, _zrow, 0)
    for r in range(ROWS_PER_TILE // ZROWS):
        pltpu.sync_copy(kd0.at[pl.ds(0, ZROWS)],
                        aggs.at[pl.ds(sid * ROWS_PER_TILE + r * ZROWS, ZROWS)])
    plsc.subcore_barrier()

    tbase = wid * CH_PER_TILE
    pltpu.async_copy(src_hbm.at[pl.ds(tbase, GRP)], srcg.at[0], semi)
    pltpu.async_copy(dst_hbm.at[pl.ds(tbase, GRP)], dstg.at[0], semi)

    kbuf = (kd0, kd1)
    wbuf = (qv0, qv1)
    gsem = (semg0, semg1)

    def _issue(gm, jj, b):
        hk = pltpu.async_copy(k_hbm.at[dstg.at[gm, jj]], kbuf[b], gsem[b])
        hq = pltpu.async_copy(qv_hbm.at[srcg.at[gm, jj]], wbuf[b], gsem[b])
        return (hk, hq)

    def _group(g, _):
        gm = lax.rem(g, 2)
        gbase = tbase + g * GRP
        pltpu.make_async_copy(src_hbm.at[pl.ds(gbase, GRP)],
                              srcg.at[gm], semi).wait()
        pltpu.make_async_copy(dst_hbm.at[pl.ds(gbase, GRP)],
                              dstg.at[gm], semi).wait()

        @pl.when(g + 1 < NGRP)
        def _():
            nb = gbase + GRP
            pltpu.async_copy(src_hbm.at[pl.ds(nb, GRP)], srcg.at[1 - gm],
                             semi)
            pltpu.async_copy(dst_hbm.at[pl.ds(nb, GRP)], dstg.at[1 - gm],
                             semi)

        pend = _issue(gm, 0, 0)
        for jj in range(GRP):
            b = jj % 2
            for hc in pend:
                hc.wait()
            if jj < GRP - 1:
                pend = _issue(gm, jj + 1, 1 - b)
            kb, wb = kbuf[b], wbuf[b]

            def _edge(e, _, kb=kb, wb=wb):
                nsl = H // 16
                eks = [kb[e, pl.ds(i * 16, 16)] for i in range(nsl)]
                wss = [wb[e, pl.ds(i * 16, 16)] for i in range(nsl)]
                eqs = [jax.lax.bitcast_convert_type(
                    jax.lax.shift_left(wss[i], 16), jnp.float32)
                    for i in range(nsl)]
                vss = [jax.lax.bitcast_convert_type(
                    jnp.bitwise_and(wss[i], jnp.int32(-65536)), jnp.float32)
                    for i in range(nsl)]
                ds = [1.0 + eks[i] * eqs[i] for i in range(nsl)]
                rs = [1.0 / ds[i] for i in range(nsl)]
                ms = [rs[i] * (2.0 - ds[i] * rs[i]) * vss[i]
                      for i in range(nsl)]
                for i in range(nsl):
                    msgv[e, pl.ds(i * 16, 16)] = ms[i]
                return 0
            lax.fori_loop(0, C, _edge, 0)

            pltpu.sync_copy(msgv, aggs.at[dstg.at[gm, jj]], add=True)
        return 0
    lax.fori_loop(0, NGRP, _group, 0)

    plsc.subcore_barrier()
    pltpu.sync_copy(aggs.at[pl.ds(sid * ROWS_PER_TILE, ROWS_PER_TILE)],
                    out_hbm.at[cid, pl.ds(sid * ROWS_PER_TILE, ROWS_PER_TILE)])


_edge_call = functools.partial(
    pl.kernel,
    mesh=plsc.VectorSubcoreMesh(core_axis_name="c", subcore_axis_name="s"),
    out_type=jax.ShapeDtypeStruct((NC, NPAD, H), jnp.float32),
    scratch_types=[
        pltpu.VMEM((2, GRP, C), jnp.int32),
        pltpu.VMEM((2, GRP, C), jnp.int32),
        pltpu.VMEM((C, H), jnp.float32),
        pltpu.VMEM((C, H), jnp.float32),
        pltpu.VMEM((C, H), jnp.int32),
        pltpu.VMEM((C, H), jnp.int32),
        pltpu.VMEM((C, H), jnp.float32),
        pltpu.VMEM_SHARED((NPAD, H), jnp.float32),
        pltpu.SemaphoreType.DMA,
        pltpu.SemaphoreType.DMA,
        pltpu.SemaphoreType.DMA,
    ],
)(_edge_body)


R = 1000
NB = N // R


def _negexp(x):
    return jnp.exp(-jnp.clip(x, -44.0, 44.0))


def _pack16(hi_f32, lo_f32):
    hi = jax.lax.bitcast_convert_type(
        hi_f32.astype(jnp.bfloat16), jnp.uint16).astype(jnp.int32)
    lo = jax.lax.bitcast_convert_type(
        lo_f32.astype(jnp.bfloat16), jnp.uint16).astype(jnp.int32)
    return jax.lax.shift_left(hi, 16) | lo


def _emit_kqvs(i, acc, k_ref, qv_ref, s_ref):
    @pl.when(i < NB)
    def _():
        k_ref[...] = _negexp(acc[:, :H])
        qv_ref[...] = _pack16(acc[:, 2 * H:3 * H],
                              _negexp(acc[:, H:2 * H]))
        s_ref[...] = acc[:, 3 * H:]

    @pl.when(i == NB)
    def _():
        k_ref[...] = jnp.ones(k_ref.shape, k_ref.dtype)
        qv_ref[...] = jnp.full(qv_ref.shape, 0x3F80, qv_ref.dtype)
        s_ref[...] = jnp.zeros(s_ref.shape, s_ref.dtype)


def _mm4_body(x_ref, w_ref, b_ref, k_ref, qv_ref, s_ref):
    i = pl.program_id(0)
    acc = _dot(x_ref[...], w_ref[...]) + b_ref[...]
    _emit_kqvs(i, acc, k_ref, qv_ref, s_ref)


def _clampi(i):
    return jnp.minimum(i, NB - 1)


_MM4_OUT_SPECS = [
    pl.BlockSpec((R, H), lambda i: (i, 0)),
    pl.BlockSpec((R, H), lambda i: (i, 0)),
    pl.BlockSpec((R, H), lambda i: (i, 0)),
]
_MM4_OUT_SHAPE = [
    jax.ShapeDtypeStruct((NPAD, H), jnp.float32),
    jax.ShapeDtypeStruct((NPAD, H), jnp.int32),
    jax.ShapeDtypeStruct((NPAD, H), jnp.float32),
]


def _mm4(x, w4, b4):
    return pl.pallas_call(
        _mm4_body,
        grid=(NB + 1,),
        in_specs=[
            pl.BlockSpec((R, H), lambda i: (_clampi(i), 0)),
            pl.BlockSpec((H, 4 * H), lambda i: (0, 0)),
            pl.BlockSpec((1, 4 * H), lambda i: (0, 0)),
        ],
        out_specs=_MM4_OUT_SPECS,
        out_shape=_MM4_OUT_SHAPE,
    )(x, w4, b4)


def _relu_stats_body(a_ref, s_ref, y_ref, st_ref):
    i = pl.program_id(0)
    y = jnp.maximum(a_ref[0] + a_ref[1] + s_ref[...], 0.0)
    y_ref[...] = y
    part = jnp.concatenate(
        [jnp.sum(y, axis=0, keepdims=True),
         jnp.sum(y * y, axis=0, keepdims=True)], axis=0)

    @pl.when(i == 0)
    def _():
        st_ref[...] = part

    @pl.when(i > 0)
    def _():
        st_ref[...] = st_ref[...] + part


def _relu_stats(agg2, s):
    return pl.pallas_call(
        _relu_stats_body,
        grid=(NB,),
        in_specs=[
            pl.BlockSpec((NC, R, H), lambda i: (0, i, 0)),
            pl.BlockSpec((R, H), lambda i: (i, 0)),
        ],
        out_specs=[
            pl.BlockSpec((R, H), lambda i: (i, 0)),
            pl.BlockSpec((2, H), lambda i: (0, 0)),
        ],
        out_shape=[
            jax.ShapeDtypeStruct((N, H), jnp.float32),
            jax.ShapeDtypeStruct((2, H), jnp.float32),
        ],
    )(agg2, s)


def _bn(y, st_ref, g_ref, b_ref):
    mu = st_ref[0:1] * (1.0 / N)
    var = st_ref[1:2] * (1.0 / N) - mu * mu
    inv = jax.lax.rsqrt(var + 1e-5)
    return (y - mu) * inv * g_ref[...] + b_ref[...]


def _bn_mm4_body(y_ref, st_ref, g_ref, b_ref, w_ref, bb_ref,
                 k_ref, qv_ref, s_ref):
    i = pl.program_id(0)
    h = _bn(y_ref[...], st_ref, g_ref, b_ref)
    acc = _dot(h, w_ref[...]) + bb_ref[...]
    _emit_kqvs(i, acc, k_ref, qv_ref, s_ref)


def _bn_mm4(y, st, g, b, w4, b4):
    return pl.pallas_call(
        _bn_mm4_body,
        grid=(NB + 1,),
        in_specs=[
            pl.BlockSpec((R, H), lambda i: (_clampi(i), 0)),
            pl.BlockSpec((2, H), lambda i: (0, 0)),
            pl.BlockSpec((1, H), lambda i: (0, 0)),
            pl.BlockSpec((1, H), lambda i: (0, 0)),
            pl.BlockSpec((H, 4 * H), lambda i: (0, 0)),
            pl.BlockSpec((1, 4 * H), lambda i: (0, 0)),
        ],
        out_specs=_MM4_OUT_SPECS,
        out_shape=_MM4_OUT_SHAPE,
    )(y, st, g, b, w4, b4)


def _bn_gsum_body(y_ref, st_ref, g_ref, b_ref, bat_ref, sum_ref, cnt_ref):
    i = pl.program_id(0)
    h = _bn(y_ref[...], st_ref, g_ref, b_ref)
    gids = jax.lax.broadcasted_iota(jnp.int32, (G, R), 0)
    onehot = jnp.where(gids == bat_ref[0], 1.0, 0.0)
    psum = _dot(onehot, h)
    pcnt = _dot(onehot, jnp.ones((R, 1), jnp.float32))

    @pl.when(i == 0)
    def _():
        sum_ref[...] = psum
        cnt_ref[...] = pcnt

    @pl.when(i > 0)
    def _():
        sum_ref[...] = sum_ref[...] + psum
        cnt_ref[...] = cnt_ref[...] + pcnt


def _bn_gsum(y, st, g, b, batch_row):
    return pl.pallas_call(
        _bn_gsum_body,
        grid=(NB,),
        in_specs=[
            pl.BlockSpec((R, H), lambda i: (i, 0)),
            pl.BlockSpec((2, H), lambda i: (0, 0)),
            pl.BlockSpec((1, H), lambda i: (0, 0)),
            pl.BlockSpec((1, H), lambda i: (0, 0)),
            pl.BlockSpec((1, 1, R), lambda i: (i, 0, 0)),
        ],
        out_specs=[
            pl.BlockSpec((G, H), lambda i: (0, 0)),
            pl.BlockSpec((G, 1), lambda i: (0, 0)),
        ],
        out_shape=[
            jax.ShapeDtypeStruct((G, H), jnp.float32),
            jax.ShapeDtypeStruct((G, 1), jnp.float32),
        ],
    )(y, st, g, b, batch_row)


def _head_body(sum_ref, cnt_ref, wft_ref, bft_ref, lng_ref, lnb_ref,
               wa1_ref, ba1_ref, wa2_ref, ba2_ref, wr_ref, br_ref, out_ref):
    xm = sum_ref[...] / jnp.maximum(cnt_ref[...], 1.0)
    t = _dot(xm, wft_ref[...]) + bft_ref[...]
    mu = jnp.mean(t, axis=1, keepdims=True)
    var = jnp.mean(t * t, axis=1, keepdims=True) - mu * mu
    t = (t - mu) * jax.lax.rsqrt(var + 1e-5) * lng_ref[...] + lnb_ref[...]
    t = jnp.maximum(t, 0.0)
    a = jnp.maximum(_dot(t, wa1_ref[...]) + ba1_ref[...], 0.0)
    att = jax.nn.sigmoid(_dot(a, wa2_ref[...]) + ba2_ref[...])
    out_ref[...] = _dot(t * att, wr_ref[...]) + br_ref[...]


def _head(sums, cnts, hp):
    return pl.pallas_call(
        _head_body,
        out_shape=jax.ShapeDtypeStruct((G, 1), jnp.float32),
    )(sums, cnts,
      hp["W_ft"], hp["b_ft"].reshape(1, H),
      hp["ln_g"].reshape(1, H), hp["ln_b"].reshape(1, H),
      hp["Wa1"], hp["ba1"].reshape(1, H // 4),
      hp["Wa2"], hp["ba2"].reshape(1, 1),
      hp["Wr"], hp["br"].reshape(1, 1))


def kernel(X, edge_index, batch, params):
    pad = N + (jnp.arange(EPAD - E, dtype=jnp.int32) % (NPAD - N))
    src2 = jnp.concatenate([edge_index[0], pad]).reshape(NT * CH_PER_TILE, C)
    dst2 = jnp.concatenate([edge_index[1], pad]).reshape(NT * CH_PER_TILE, C)
    batch_row = batch.reshape(NB, 1, R)

    w4, b4 = [], []
    for cv in params["convs"]:
        w4.append(jnp.concatenate(
            [cv["Wk"], cv["Wq"], cv["Wv"], cv["Ws"]], axis=1))
        b4.append(jnp.concatenate(
            [cv["bk"], cv["bq"], cv["bv"], cv["bs"]]).reshape(1, 4 * H))
    gam = [b["gamma"].reshape(1, H) for b in params["bns"]]
    bet = [b["beta"].reshape(1, H) for b in params["bns"]]

    ek, qvw, s = _mm4(X, w4[0], b4[0])
    for i in range(5):
        agg2 = _edge_call(ek, qvw, src2, dst2)
        y, st = _relu_stats(agg2, s)
        if i < 4:
            ek, qvw, s = _bn_mm4(y, st, gam[i], bet[i], w4[i + 1], b4[i + 1])
        else:
            sums, cnts = _bn_gsum(y, st, gam[i], bet[i], batch_row)
    return _head(sums, cnts, params["head"])
